# baseline (speedup 1.0000x reference)
import jax
import jax.numpy as jnp
from jax import lax
from jax.experimental import pallas as pl
from jax.experimental.pallas import tpu as pltpu
from jax.experimental.pallas import tpu_sc as plsc

D_MODEL = 128
BATCH = 132
LANES = 16
NUM_TILES = 9
CHUNKS = D_MODEL // LANES
IDX_PER_TILE = 6 * LANES
TAIL_STORE = 128

_COLS = (0, 2, 3, 4, 5, 1)
_SIZES = (20, 4, 8, 8, 3, 100001)
_OFFS = (0, 20, 24, 32, 40, 0)


def _sc_body(sidx_hbm, stacked_hbm, card_hbm, out_hbm, sblk_v, gath5_v,
             gathc_v, acc_v, sem):
    wid = lax.axis_index("s")

    @pl.when(wid < NUM_TILES)
    def _():
        pltpu.sync_copy(sidx_hbm.at[pl.ds(wid * IDX_PER_TILE, IDX_PER_TILE)],
                        sblk_v)
        cp1 = pltpu.async_copy(stacked_hbm.at[sblk_v.at[pl.ds(0, 5 * LANES)]],
                               gath5_v, sem)
        cp2 = pltpu.async_copy(card_hbm.at[sblk_v.at[pl.ds(5 * LANES, LANES)]],
                               gathc_v, sem)
        cp1.wait()
        cp2.wait()

        def row(i, _):
            for c in range(CHUNKS):
                sl = pl.ds(c * LANES, LANES)
                acc_v[i, sl] = (
                    gath5_v[i, sl]
                    + gath5_v[LANES + i, sl]
                    + gath5_v[2 * LANES + i, sl]
                    + gath5_v[3 * LANES + i, sl]
                    + gath5_v[4 * LANES + i, sl]
                    + gathc_v[i, sl]
                )
            return 0

        lax.fori_loop(0, LANES, row, 0)

        @pl.when(wid < NUM_TILES - 1)
        def _():
            off = pl.multiple_of(wid * LANES, 8)
            pltpu.sync_copy(acc_v, out_hbm.at[pl.ds(off, LANES)])

        @pl.when(wid == NUM_TILES - 1)
        def _():
            pltpu.sync_copy(
                acc_v.at[pl.ds(0, BATCH - TAIL_STORE)],
                out_hbm.at[pl.ds(TAIL_STORE, BATCH - TAIL_STORE)],
            )


@jax.jit
def _sc_embed(sidx, stacked, card_table):
    mesh = plsc.VectorSubcoreMesh(core_axis_name="c", subcore_axis_name="s",
                                  num_cores=1)
    return pl.kernel(
        _sc_body,
        out_type=jax.ShapeDtypeStruct((BATCH, D_MODEL), jnp.float32),
        mesh=mesh,
        scratch_types=[
            pltpu.VMEM((IDX_PER_TILE,), jnp.int32),
            pltpu.VMEM((5 * LANES, D_MODEL), jnp.float32),
            pltpu.VMEM((LANES, D_MODEL), jnp.float32),
            pltpu.VMEM((LANES, D_MODEL), jnp.float32),
            pltpu.SemaphoreType.DMA,
        ],
        compiler_params=pltpu.CompilerParams(needs_layout_passes=False),
    )(sidx, stacked, card_table)


def kernel(x, turn_table, card_table, action_table, pos_table, civ_table,
           face_table):
    x = x.astype(jnp.int32)
    stacked = jnp.concatenate(
        [turn_table, action_table, pos_table, civ_table, face_table], axis=0)
    rows = jnp.minimum(jnp.arange(NUM_TILES * LANES, dtype=jnp.int32),
                       BATCH - 1)
    cols = []
    for c, sz, off in zip(_COLS, _SIZES, _OFFS):
        cols.append(jnp.clip(x[rows, c], 0, sz - 1) + off)
    sidx = jnp.stack(cols).reshape(6, NUM_TILES, LANES).transpose(1, 0, 2)
    sidx = sidx.reshape(NUM_TILES * IDX_PER_TILE)
    return _sc_embed(sidx, stacked, card_table)


# speedup vs baseline: 1.6987x; 1.6987x over previous
import jax
import jax.numpy as jnp
from jax import lax
from jax.experimental import pallas as pl
from jax.experimental.pallas import tpu as pltpu
from jax.experimental.pallas import tpu_sc as plsc

D_MODEL = 128
BATCH = 132
B_PAD = 136
NUM_TABLES = 6
LANES = 16
NUM_TILES = 9
CHUNKS = D_MODEL // LANES
LAST_BASE = 120
TAIL_STORE = 128


def _sc_body(xtf_hbm, t0, t1, t2, t3, t4, t5, out_hbm, idx_v, gath_v, acc_v,
             sem):
    wid = lax.axis_index("s")
    tables = (t0, t1, t2, t3, t4, t5)

    @pl.when(wid < NUM_TILES)
    def _():
        base = pl.multiple_of(jnp.minimum(wid * LANES, LAST_BASE), 8)
        for t in range(NUM_TABLES):
            pltpu.sync_copy(xtf_hbm.at[pl.ds(t * B_PAD + base, LANES)],
                            idx_v.at[t])
        copies = []
        for t in range(NUM_TABLES):
            copies.append(
                pltpu.async_copy(tables[t].at[idx_v.at[t]], gath_v.at[t], sem)
            )
        for cp in copies:
            cp.wait()

        def row(i, _):
            for c in range(CHUNKS):
                sl = pl.ds(c * LANES, LANES)
                acc_v[i, sl] = (
                    gath_v[0, i, sl]
                    + gath_v[1, i, sl]
                    + gath_v[2, i, sl]
                    + gath_v[3, i, sl]
                    + gath_v[4, i, sl]
                    + gath_v[5, i, sl]
                )
            return 0

        lax.fori_loop(0, LANES, row, 0)

        @pl.when(wid < NUM_TILES - 1)
        def _():
            off = pl.multiple_of(wid * LANES, 8)
            pltpu.sync_copy(acc_v, out_hbm.at[pl.ds(off, LANES)])

        @pl.when(wid == NUM_TILES - 1)
        def _():
            pltpu.sync_copy(
                acc_v.at[pl.ds(TAIL_STORE - LAST_BASE, BATCH - TAIL_STORE)],
                out_hbm.at[pl.ds(TAIL_STORE, BATCH - TAIL_STORE)],
            )


@jax.jit
def _sc_embed(xtf, turn_table, card_table, action_table, pos_table, civ_table,
              face_table):
    mesh = plsc.VectorSubcoreMesh(core_axis_name="c", subcore_axis_name="s",
                                  num_cores=1)
    return pl.kernel(
        _sc_body,
        out_type=jax.ShapeDtypeStruct((BATCH, D_MODEL), jnp.float32),
        mesh=mesh,
        scratch_types=[
            pltpu.VMEM((NUM_TABLES, LANES), jnp.int32),
            pltpu.VMEM((NUM_TABLES, LANES, D_MODEL), jnp.float32),
            pltpu.VMEM((LANES, D_MODEL), jnp.float32),
            pltpu.SemaphoreType.DMA,
        ],
    )(xtf, turn_table, card_table, action_table, pos_table, civ_table,
      face_table)


def kernel(x, turn_table, card_table, action_table, pos_table, civ_table,
           face_table):
    xt = jnp.transpose(x.astype(jnp.int32))
    xt = jnp.pad(xt, ((0, 0), (0, B_PAD - BATCH)))
    return _sc_embed(xt.reshape(NUM_TABLES * B_PAD), turn_table, card_table,
                     action_table, pos_table, civ_table, face_table)


# speedup vs baseline: 1.9529x; 1.1497x over previous
import jax
import jax.numpy as jnp
from jax import lax
from jax.experimental import pallas as pl
from jax.experimental.pallas import tpu as pltpu
from jax.experimental.pallas import tpu_sc as plsc

D_MODEL = 128
BATCH = 132
NUM_TABLES = 6
LANES = 16
NUM_TILES = 9
CHUNKS = D_MODEL // LANES
LAST_BASE = BATCH - LANES
TAIL_STORE = 128


def _sc_body(xf_hbm, t0, t1, t2, t3, t4, t5, out_hbm, xblk_v, idx_v, gath_v,
             acc_v, sem):
    wid = lax.axis_index("s")
    tables = (t0, t1, t2, t3, t4, t5)

    @pl.when(wid < NUM_TILES)
    def _():
        base_row = jnp.minimum(wid * LANES, LAST_BASE)
        pltpu.sync_copy(
            xf_hbm.at[pl.ds(base_row * NUM_TABLES, LANES * NUM_TABLES)],
            xblk_v,
        )
        lane = lax.iota(jnp.int32, LANES)
        for t in range(NUM_TABLES):
            idx_v[t, :] = plsc.load_gather(xblk_v, [lane * NUM_TABLES + t])
        copies = []
        for t in range(NUM_TABLES):
            copies.append(
                pltpu.async_copy(tables[t].at[idx_v.at[t]], gath_v.at[t], sem)
            )
        for cp in copies:
            cp.wait()

        def row(i, _):
            for c in range(CHUNKS):
                sl = pl.ds(c * LANES, LANES)
                acc_v[i, sl] = (
                    gath_v[0, i, sl]
                    + gath_v[1, i, sl]
                    + gath_v[2, i, sl]
                    + gath_v[3, i, sl]
                    + gath_v[4, i, sl]
                    + gath_v[5, i, sl]
                )
            return 0

        lax.fori_loop(0, LANES, row, 0)

        @pl.when(wid < NUM_TILES - 1)
        def _():
            off = pl.multiple_of(wid * LANES, 8)
            pltpu.sync_copy(acc_v, out_hbm.at[pl.ds(off, LANES)])

        @pl.when(wid == NUM_TILES - 1)
        def _():
            pltpu.sync_copy(
                acc_v.at[pl.ds(TAIL_STORE - LAST_BASE, BATCH - TAIL_STORE)],
                out_hbm.at[pl.ds(TAIL_STORE, BATCH - TAIL_STORE)],
            )


@jax.jit
def _sc_embed(xf, turn_table, card_table, action_table, pos_table, civ_table,
              face_table):
    mesh = plsc.VectorSubcoreMesh(core_axis_name="c", subcore_axis_name="s",
                                  num_cores=1)
    return pl.kernel(
        _sc_body,
        out_type=jax.ShapeDtypeStruct((BATCH, D_MODEL), jnp.float32),
        mesh=mesh,
        scratch_types=[
            pltpu.VMEM((LANES * NUM_TABLES,), jnp.int32),
            pltpu.VMEM((NUM_TABLES, LANES), jnp.int32),
            pltpu.VMEM((NUM_TABLES, LANES, D_MODEL), jnp.float32),
            pltpu.VMEM((LANES, D_MODEL), jnp.float32),
            pltpu.SemaphoreType.DMA,
        ],
        compiler_params=pltpu.CompilerParams(needs_layout_passes=False),
    )(xf, turn_table, card_table, action_table, pos_table, civ_table,
      face_table)


def kernel(x, turn_table, card_table, action_table, pos_table, civ_table,
           face_table):
    xf = jnp.reshape(x.astype(jnp.int32), (-1,))
    return _sc_embed(xf, turn_table, card_table, action_table, pos_table,
                     civ_table, face_table)
